# baseline (speedup 1.0000x reference)
import functools

import jax
import jax.numpy as jnp
from jax import lax
from jax.experimental import pallas as pl
from jax.experimental.pallas import tpu as pltpu
from jax.experimental.pallas import tpu_sc as plsc


def _ln(r, scale, bias, eps=1e-5):
    mu = jnp.mean(r, axis=-1, keepdims=True)
    var = jnp.mean((r - mu) ** 2, axis=-1, keepdims=True)
    return (r - mu) * lax.rsqrt(var + eps) * scale + bias


def _pack_bf16_halves(m, dh):
    bits = lax.bitcast_convert_type(m, jnp.uint32)

    def rtne_hi(b):
        return (b + (0x7FFF + ((b >> 16) & 1))) & jnp.uint32(0xFFFF0000)

    lo = rtne_hi(bits[:, :dh]) >> 16
    hi = rtne_hi(bits[:, dh:])
    return lax.bitcast_convert_type(hi | lo, jnp.float32)


def _unpack_bf16_halves(p):
    w = lax.bitcast_convert_type(p, jnp.uint32)
    lo = lax.bitcast_convert_type(w << 16, jnp.float32)
    hi = lax.bitcast_convert_type(w & jnp.uint32(0xFFFF0000), jnp.float32)
    return jnp.concatenate([lo, hi], axis=-1)


def _precompute_tables(x, w_sd):
    n, d = x.shape
    bn = 2000
    assert n % bn == 0

    def body(x_ref, w_ref, ps_ref, pd_ref):
        xb = x_ref[...]
        w = w_ref[...]
        xb16 = xb.astype(jnp.bfloat16)
        w16 = w.astype(jnp.bfloat16)
        ps = jnp.dot(xb16, w16[:d], preferred_element_type=jnp.float32)
        pd = jnp.dot(xb16, w16[d:], preferred_element_type=jnp.float32)
        ps_ref[...] = _pack_bf16_halves(ps, d // 2)
        pd_ref[...] = _pack_bf16_halves(pd, d // 2)

    return pl.pallas_call(
        body,
        grid=(n // bn,),
        in_specs=[
            pl.BlockSpec((bn, d), lambda i: (i, 0)),
            pl.BlockSpec((2 * d, d), lambda i: (0, 0)),
        ],
        out_specs=[
            pl.BlockSpec((bn, d // 2), lambda i: (i, 0)),
            pl.BlockSpec((bn, d // 2), lambda i: (i, 0)),
        ],
        out_shape=[
            jax.ShapeDtypeStruct((n, d // 2), jnp.float32),
            jax.ShapeDtypeStruct((n, d // 2), jnp.float32),
        ],
    )(x, w_sd)


def _sc_gather_pair(src, dst, ps, pd, window):
    e_off, e_len = window
    _, d = ps.shape
    info = plsc.get_sparse_core_info()
    nc, ns = info.num_cores, info.num_subcores
    nw = nc * ns
    ew = e_len // nw
    ch = 200
    nfull = (ew // ch) & ~1
    tail = ew - nfull * ch
    assert e_len % nw == 0 and ew % 8 == 0 and tail % 8 == 0 and tail <= ch

    mesh = plsc.VectorSubcoreMesh(core_axis_name="c", subcore_axis_name="s")

    @functools.partial(
        pl.kernel,
        out_type=(
            jax.ShapeDtypeStruct((e_len, d), jnp.float32),
            jax.ShapeDtypeStruct((e_len, d), jnp.float32),
        ),
        mesh=mesh,
        scratch_types=[
            pltpu.VMEM((ew,), jnp.int32),
            pltpu.VMEM((ew,), jnp.int32),
            [pltpu.VMEM((ch, d), jnp.float32) for _ in range(2)],
            [pltpu.VMEM((ch, d), jnp.float32) for _ in range(2)],
            [pltpu.SemaphoreType.DMA for _ in range(2)],
            [pltpu.SemaphoreType.DMA for _ in range(2)],
        ],
    )
    def k(src_hbm, dst_hbm, ps_hbm, pd_hbm, g1_hbm, g2_hbm,
          idx_s, idx_d, bufs_a, bufs_b, sems_g, sems_w):
        wid = lax.axis_index("s") * nc + lax.axis_index("c")
        base = wid * ew
        pltpu.sync_copy(src_hbm.at[pl.ds(e_off + base, ew)], idx_s)
        pltpu.sync_copy(dst_hbm.at[pl.ds(e_off + base, ew)], idx_d)

        def fire_gather(i, b, n):
            pltpu.async_copy(ps_hbm.at[idx_s.at[pl.ds(i * ch, n)]],
                             bufs_a[b].at[pl.ds(0, n)], sems_g[b])
            pltpu.async_copy(pd_hbm.at[idx_d.at[pl.ds(i * ch, n)]],
                             bufs_b[b].at[pl.ds(0, n)], sems_g[b])

        def wait_gather(i, b, n):
            pltpu.make_async_copy(ps_hbm.at[idx_s.at[pl.ds(i * ch, n)]],
                                  bufs_a[b].at[pl.ds(0, n)], sems_g[b]).wait()
            pltpu.make_async_copy(pd_hbm.at[idx_d.at[pl.ds(i * ch, n)]],
                                  bufs_b[b].at[pl.ds(0, n)], sems_g[b]).wait()

        def fire_wb(i, b, n):
            row0 = base + i * ch
            pltpu.async_copy(bufs_a[b].at[pl.ds(0, n)],
                             g1_hbm.at[pl.ds(row0, n)], sems_w[b])
            pltpu.async_copy(bufs_b[b].at[pl.ds(0, n)],
                             g2_hbm.at[pl.ds(row0, n)], sems_w[b])

        def wait_wb(i, b, n):
            row0 = base + i * ch
            pltpu.make_async_copy(bufs_a[b].at[pl.ds(0, n)],
                                  g1_hbm.at[pl.ds(row0, n)], sems_w[b]).wait()
            pltpu.make_async_copy(bufs_b[b].at[pl.ds(0, n)],
                                  g2_hbm.at[pl.ds(row0, n)], sems_w[b]).wait()

        @pl.loop(0, nfull // 2)
        def _(g):
            for b in (0, 1):
                i = g * 2 + b

                @pl.when(i >= 2)
                def _():
                    wait_wb(i - 2, b, ch)

                fire_gather(i, b, ch)
            for b in (0, 1):
                i = g * 2 + b
                wait_gather(i, b, ch)
                fire_wb(i, b, ch)

        if nfull >= 2:
            wait_wb(nfull - 2, 0, ch)
            wait_wb(nfull - 1, 1, ch)
        if tail:
            fire_gather(nfull, 0, tail)
            wait_gather(nfull, 0, tail)
            fire_wb(nfull, 0, tail)
            wait_wb(nfull, 0, tail)

    return k(src, dst, ps, pd)


def _edge_mlp(g1, g2, edge_attr, w_e, b_e1, w_e2, b_e2, ln_s, ln_b,
              window, eo_seed=None):
    e, d = edge_attr.shape
    e_off, e_len = window
    be = 2000
    assert e_len % be == 0 and e_off % be == 0
    boff = e_off // be

    def body(g1_ref, g2_ref, ea_ref, we_ref, b1_ref, w2_ref, b2_ref,
             s_ref, b_ref, *rest):
        u_ref, eo_ref = rest[-2], rest[-1]
        ea = ea_ref[...]
        z = (_unpack_bf16_halves(g1_ref[...]) + _unpack_bf16_halves(g2_ref[...])
             + b1_ref[...]
             + jnp.dot(ea.astype(jnp.bfloat16),
                       we_ref[...].astype(jnp.bfloat16),
                       preferred_element_type=jnp.float32))
        t = jnp.maximum(z, 0.0)
        u = jnp.dot(t.astype(jnp.bfloat16),
                    w2_ref[...].astype(jnp.bfloat16),
                    preferred_element_type=jnp.float32) + b2_ref[...]
        u_ref[...] = u
        eo_ref[...] = _ln(u + ea, s_ref[...], b_ref[...])

    vec = lambda: pl.BlockSpec((1, d), lambda i: (0, 0))
    mat = lambda: pl.BlockSpec((d, d), lambda i: (0, 0))
    blk = lambda: pl.BlockSpec((be, d), lambda i: (i, 0))
    out_blk = lambda: pl.BlockSpec((be, d), lambda i: (i + boff, 0))
    hlf = lambda: pl.BlockSpec((be, d // 2), lambda i: (i, 0))
    we_spec = pl.BlockSpec((d, d), lambda i: (2, 0))
    in_specs = [hlf(), hlf(), out_blk(), we_spec, vec(), mat(), vec(), vec(),
                vec()]
    args = [g1, g2, edge_attr, w_e, b_e1.reshape(1, d), w_e2,
            b_e2.reshape(1, d), ln_s.reshape(1, d), ln_b.reshape(1, d)]
    io_aliases = {}
    if eo_seed is not None:
        io_aliases[len(args)] = 1
        in_specs.append(pl.BlockSpec(memory_space=pl.ANY))
        args.append(eo_seed)
    return pl.pallas_call(
        body,
        grid=(e_len // be,),
        in_specs=in_specs,
        out_specs=[blk(), out_blk()],
        out_shape=[
            jax.ShapeDtypeStruct((e_len, d), jnp.float32),
            jax.ShapeDtypeStruct((e, d), jnp.float32),
        ],
        input_output_aliases=io_aliases,
    )(*args)


def _sc_segment_sum(dst3, u, zeros_half):
    e, d = u.shape
    n, dh = zeros_half.shape
    assert dh == d // 2
    info = plsc.get_sparse_core_info()
    nc, ns = info.num_cores, info.num_subcores
    et = e // ns
    ch = 80
    nit = et // ch
    assert dst3.shape == (ns, nit, ch)
    rc = 400
    nrc = n // rc
    nround = (nrc + ns - 1) // ns
    assert et % ch == 0 and n % rc == 0

    mesh = plsc.VectorSubcoreMesh(core_axis_name="c", subcore_axis_name="s")

    @functools.partial(
        pl.kernel,
        out_type=jax.ShapeDtypeStruct((n, d), jnp.float32),
        mesh=mesh,
        scratch_types=[
            pltpu.VMEM((nit, ch), jnp.int32),
            [pltpu.VMEM((ch, dh), jnp.float32) for _ in range(2)],
            [pltpu.SemaphoreType.DMA for _ in range(2)],
            pltpu.MemorySpace.VMEM_SHARED((n, dh), jnp.float32),
        ],
    )
    def k(dst2_hbm, u_hbm, z_hbm, agg_hbm, idx2, rows, sems, acc):
        cid = lax.axis_index("c")
        sid = lax.axis_index("s")
        col0 = cid * dh

        def fire_rows(i, b):
            pltpu.async_copy(
                u_hbm.at[pl.ds(sid * et + i * ch, ch), pl.ds(col0, dh)],
                rows[b], sems[b])

        def wait_rows(i, b):
            pltpu.make_async_copy(
                u_hbm.at[pl.ds(sid * et + i * ch, ch), pl.ds(col0, dh)],
                rows[b], sems[b]).wait()

        pltpu.sync_copy(dst2_hbm.at[sid], idx2)

        for r in range(nround):
            j = r * ns + sid

            @pl.when(j < nrc)
            def _():
                pltpu.sync_copy(z_hbm.at[pl.ds(j * rc, rc)],
                                acc.at[pl.ds(j * rc, rc)])

        fire_rows(0, 0)
        plsc.subcore_barrier()

        @pl.loop(0, nit // 2)
        def _(g):
            for b in (0, 1):
                i = g * 2 + b

                @pl.when(i + 1 < nit)
                def _():
                    fire_rows(i + 1, 1 - b)

                wait_rows(i, b)
                pltpu.sync_copy(rows[b], acc.at[idx2.at[i]], add=True)

        if nit % 2:
            i = nit - 1
            wait_rows(i, i % 2)
            pltpu.sync_copy(rows[i % 2], acc.at[idx2.at[i]], add=True)

        plsc.subcore_barrier()

        for r in range(nround):
            j = r * ns + sid

            @pl.when(j < nrc)
            def _():
                pltpu.sync_copy(acc.at[pl.ds(j * rc, rc)],
                                agg_hbm.at[pl.ds(j * rc, rc), pl.ds(col0, dh)])

    return k(dst3, u, zeros_half)


def _node_mlp(x, aggs, w_n1, b_n1, w_n2, b_n2, ln_s, ln_b):
    n, d = x.shape
    bn = 1000
    assert n % bn == 0

    def body(x_ref, *refs):
        a_refs, (w1_ref, b1_ref, w2_ref, b2_ref, s_ref, b_ref, o_ref) = (
            refs[:-7], refs[-7:])
        xb = x_ref[...]
        agg = a_refs[0][...]
        for a in a_refs[1:]:
            agg = agg + a[...]
        w1 = w1_ref[...].astype(jnp.bfloat16)
        z = (jnp.dot(xb.astype(jnp.bfloat16), w1[:d],
                     preferred_element_type=jnp.float32)
             + jnp.dot(agg.astype(jnp.bfloat16), w1[d:],
                       preferred_element_type=jnp.float32)
             + b1_ref[...])
        t = jnp.maximum(z, 0.0)
        u = jnp.dot(t.astype(jnp.bfloat16), w2_ref[...].astype(jnp.bfloat16),
                    preferred_element_type=jnp.float32) + b2_ref[...]
        o_ref[...] = _ln(u + xb, s_ref[...], b_ref[...])

    vec = lambda: pl.BlockSpec((1, d), lambda i: (0, 0))
    blk = lambda: pl.BlockSpec((bn, d), lambda i: (i, 0))
    return pl.pallas_call(
        body,
        grid=(n // bn,),
        in_specs=[blk()] + [blk() for _ in aggs]
                 + [pl.BlockSpec((2 * d, d), lambda i: (0, 0)), vec(),
                    pl.BlockSpec((d, d), lambda i: (0, 0)), vec(), vec(),
                    vec()],
        out_specs=blk(),
        out_shape=jax.ShapeDtypeStruct((n, d), jnp.float32),
    )(x, *aggs, w_n1, b_n1.reshape(1, d), w_n2, b_n2.reshape(1, d),
      ln_s.reshape(1, d), ln_b.reshape(1, d))


def kernel(x, edge_attr, W_e1, b_e1, W_e2, b_e2, W_n1, b_n1, W_n2, b_n2,
           ln_n_scale, ln_n_bias, ln_e_scale, ln_e_bias, edge_index):
    n, d = x.shape
    src = edge_index[0]
    dst = edge_index[1]

    e = src.shape[0]
    w32 = e // 5
    windows = [(0, w32), (w32, w32), (2 * w32, 2 * w32), (4 * w32, w32)]

    sch = 80
    zeros_half = jnp.zeros((n, d // 2), jnp.float32)
    we, be1 = W_e1, b_e1

    ps, pd = _precompute_tables(x, W_e1)
    gs = [_sc_gather_pair(src, dst, ps, pd, w) for w in windows[:2]]
    eo = None
    aggs = []
    for k, w in enumerate(windows):
        u, eo = _edge_mlp(gs[k][0], gs[k][1], edge_attr, we, be1, W_e2, b_e2,
                          ln_e_scale, ln_e_bias, w, eo_seed=eo)
        if len(gs) < len(windows):
            gs.append(_sc_gather_pair(src, dst, ps, pd, windows[len(gs)]))
        dst3 = dst[w[0] : w[0] + w[1]].reshape(16, -1, sch)
        aggs.append(_sc_segment_sum(dst3, u, zeros_half))
    x_out = _node_mlp(x, aggs, W_n1, b_n1, W_n2, b_n2, ln_n_scale, ln_n_bias)
    return x_out, eo


# speedup vs baseline: 1.0494x; 1.0494x over previous
import functools

import jax
import jax.numpy as jnp
from jax import lax
from jax.experimental import pallas as pl
from jax.experimental.pallas import tpu as pltpu
from jax.experimental.pallas import tpu_sc as plsc


def _ln(r, scale, bias, eps=1e-5):
    mu = jnp.mean(r, axis=-1, keepdims=True)
    var = jnp.mean((r - mu) ** 2, axis=-1, keepdims=True)
    return (r - mu) * lax.rsqrt(var + eps) * scale + bias


def _pack_bf16_halves(m, dh):
    bits = lax.bitcast_convert_type(m, jnp.uint32)

    def rtne_hi(b):
        return (b + (0x7FFF + ((b >> 16) & 1))) & jnp.uint32(0xFFFF0000)

    lo = rtne_hi(bits[:, :dh]) >> 16
    hi = rtne_hi(bits[:, dh:])
    return lax.bitcast_convert_type(hi | lo, jnp.float32)


def _unpack_bf16_halves(p):
    w = lax.bitcast_convert_type(p, jnp.uint32)
    lo = lax.bitcast_convert_type(w << 16, jnp.float32)
    hi = lax.bitcast_convert_type(w & jnp.uint32(0xFFFF0000), jnp.float32)
    return jnp.concatenate([lo, hi], axis=-1)


def _precompute_tables(x, w_sd):
    n, d = x.shape
    bn = 2000
    assert n % bn == 0

    def body(x_ref, w_ref, ps_ref, pd_ref):
        xb = x_ref[...]
        w = w_ref[...]
        xb16 = xb.astype(jnp.bfloat16)
        w16 = w.astype(jnp.bfloat16)
        ps = jnp.dot(xb16, w16[:d], preferred_element_type=jnp.float32)
        pd = jnp.dot(xb16, w16[d:], preferred_element_type=jnp.float32)
        ps_ref[...] = _pack_bf16_halves(ps, d // 2)
        pd_ref[...] = _pack_bf16_halves(pd, d // 2)

    return pl.pallas_call(
        body,
        grid=(n // bn,),
        in_specs=[
            pl.BlockSpec((bn, d), lambda i: (i, 0)),
            pl.BlockSpec((2 * d, d), lambda i: (0, 0)),
        ],
        out_specs=[
            pl.BlockSpec((bn, d // 2), lambda i: (i, 0)),
            pl.BlockSpec((bn, d // 2), lambda i: (i, 0)),
        ],
        out_shape=[
            jax.ShapeDtypeStruct((n, d // 2), jnp.float32),
            jax.ShapeDtypeStruct((n, d // 2), jnp.float32),
        ],
    )(x, w_sd)


def _sc_gather_pair(src, dst, ps, pd, window):
    e_off, e_len = window
    _, d = ps.shape
    info = plsc.get_sparse_core_info()
    nc, ns = info.num_cores, info.num_subcores
    nw = nc * ns
    ew = e_len // nw
    ch = 200
    nfull = (ew // ch) & ~1
    tail = ew - nfull * ch
    assert e_len % nw == 0 and ew % 8 == 0 and tail % 8 == 0 and tail <= ch

    mesh = plsc.VectorSubcoreMesh(core_axis_name="c", subcore_axis_name="s")

    @functools.partial(
        pl.kernel,
        out_type=(
            jax.ShapeDtypeStruct((e_len, d), jnp.float32),
            jax.ShapeDtypeStruct((e_len, d), jnp.float32),
        ),
        mesh=mesh,
        scratch_types=[
            pltpu.VMEM((ew,), jnp.int32),
            pltpu.VMEM((ew,), jnp.int32),
            [pltpu.VMEM((ch, d), jnp.float32) for _ in range(2)],
            [pltpu.VMEM((ch, d), jnp.float32) for _ in range(2)],
            [pltpu.SemaphoreType.DMA for _ in range(2)],
            [pltpu.SemaphoreType.DMA for _ in range(2)],
        ],
    )
    def k(src_hbm, dst_hbm, ps_hbm, pd_hbm, g1_hbm, g2_hbm,
          idx_s, idx_d, bufs_a, bufs_b, sems_g, sems_w):
        wid = lax.axis_index("s") * nc + lax.axis_index("c")
        base = wid * ew
        pltpu.sync_copy(src_hbm.at[pl.ds(e_off + base, ew)], idx_s)
        pltpu.sync_copy(dst_hbm.at[pl.ds(e_off + base, ew)], idx_d)

        def fire_gather(i, b, n):
            pltpu.async_copy(ps_hbm.at[idx_s.at[pl.ds(i * ch, n)]],
                             bufs_a[b].at[pl.ds(0, n)], sems_g[b])
            pltpu.async_copy(pd_hbm.at[idx_d.at[pl.ds(i * ch, n)]],
                             bufs_b[b].at[pl.ds(0, n)], sems_g[b])

        def wait_gather(i, b, n):
            pltpu.make_async_copy(ps_hbm.at[idx_s.at[pl.ds(i * ch, n)]],
                                  bufs_a[b].at[pl.ds(0, n)], sems_g[b]).wait()
            pltpu.make_async_copy(pd_hbm.at[idx_d.at[pl.ds(i * ch, n)]],
                                  bufs_b[b].at[pl.ds(0, n)], sems_g[b]).wait()

        def fire_wb(i, b, n):
            row0 = base + i * ch
            pltpu.async_copy(bufs_a[b].at[pl.ds(0, n)],
                             g1_hbm.at[pl.ds(row0, n)], sems_w[b])
            pltpu.async_copy(bufs_b[b].at[pl.ds(0, n)],
                             g2_hbm.at[pl.ds(row0, n)], sems_w[b])

        def wait_wb(i, b, n):
            row0 = base + i * ch
            pltpu.make_async_copy(bufs_a[b].at[pl.ds(0, n)],
                                  g1_hbm.at[pl.ds(row0, n)], sems_w[b]).wait()
            pltpu.make_async_copy(bufs_b[b].at[pl.ds(0, n)],
                                  g2_hbm.at[pl.ds(row0, n)], sems_w[b]).wait()

        @pl.loop(0, nfull // 2)
        def _(g):
            for b in (0, 1):
                i = g * 2 + b

                @pl.when(i >= 2)
                def _():
                    wait_wb(i - 2, b, ch)

                fire_gather(i, b, ch)
            for b in (0, 1):
                i = g * 2 + b
                wait_gather(i, b, ch)
                fire_wb(i, b, ch)

        if nfull >= 2:
            wait_wb(nfull - 2, 0, ch)
            wait_wb(nfull - 1, 1, ch)
        if tail:
            fire_gather(nfull, 0, tail)
            wait_gather(nfull, 0, tail)
            fire_wb(nfull, 0, tail)
            wait_wb(nfull, 0, tail)

    return k(src, dst, ps, pd)


def _edge_mlp(g1, g2, edge_attr, w_e, b_e1, w_e2, b_e2, ln_s, ln_b,
              window, eo_seed=None):
    e, d = edge_attr.shape
    e_off, e_len = window
    be = 2000
    assert e_len % be == 0 and e_off % be == 0
    boff = e_off // be

    def body(g1_ref, g2_ref, ea_ref, we_ref, b1_ref, w2_ref, b2_ref,
             s_ref, b_ref, *rest):
        u_ref, eo_ref = rest[-2], rest[-1]
        ea = ea_ref[...]
        z = (_unpack_bf16_halves(g1_ref[...]) + _unpack_bf16_halves(g2_ref[...])
             + b1_ref[...]
             + jnp.dot(ea.astype(jnp.bfloat16),
                       we_ref[...].astype(jnp.bfloat16),
                       preferred_element_type=jnp.float32))
        t = jnp.maximum(z, 0.0)
        u = jnp.dot(t.astype(jnp.bfloat16),
                    w2_ref[...].astype(jnp.bfloat16),
                    preferred_element_type=jnp.float32) + b2_ref[...]
        u_ref[...] = u
        eo_ref[...] = _ln(u + ea, s_ref[...], b_ref[...])

    vec = lambda: pl.BlockSpec((1, d), lambda i: (0, 0))
    mat = lambda: pl.BlockSpec((d, d), lambda i: (0, 0))
    blk = lambda: pl.BlockSpec((be, d), lambda i: (i, 0))
    out_blk = lambda: pl.BlockSpec((be, d), lambda i: (i + boff, 0))
    hlf = lambda: pl.BlockSpec((be, d // 2), lambda i: (i, 0))
    we_spec = pl.BlockSpec((d, d), lambda i: (2, 0))
    in_specs = [hlf(), hlf(), out_blk(), we_spec, vec(), mat(), vec(), vec(),
                vec()]
    args = [g1, g2, edge_attr, w_e, b_e1.reshape(1, d), w_e2,
            b_e2.reshape(1, d), ln_s.reshape(1, d), ln_b.reshape(1, d)]
    io_aliases = {}
    if eo_seed is not None:
        io_aliases[len(args)] = 1
        in_specs.append(pl.BlockSpec(memory_space=pl.ANY))
        args.append(eo_seed)
    return pl.pallas_call(
        body,
        grid=(e_len // be,),
        in_specs=in_specs,
        out_specs=[blk(), out_blk()],
        out_shape=[
            jax.ShapeDtypeStruct((e_len, d), jnp.float32),
            jax.ShapeDtypeStruct((e, d), jnp.float32),
        ],
        input_output_aliases=io_aliases,
    )(*args)


def _sc_segment_sum(dst3, u, zeros_half):
    e, d = u.shape
    n, dh = zeros_half.shape
    assert dh == d // 2
    info = plsc.get_sparse_core_info()
    nc, ns = info.num_cores, info.num_subcores
    et = e // ns
    ch = 80
    nit = et // ch
    assert dst3.shape == (ns, nit, ch)
    rc = 400
    nrc = n // rc
    nround = (nrc + ns - 1) // ns
    assert et % ch == 0 and n % rc == 0

    mesh = plsc.VectorSubcoreMesh(core_axis_name="c", subcore_axis_name="s")

    @functools.partial(
        pl.kernel,
        out_type=jax.ShapeDtypeStruct((n, d), jnp.float32),
        mesh=mesh,
        scratch_types=[
            pltpu.VMEM((nit, ch), jnp.int32),
            [pltpu.VMEM((ch, dh), jnp.float32) for _ in range(2)],
            [pltpu.SemaphoreType.DMA for _ in range(2)],
            pltpu.MemorySpace.VMEM_SHARED((n, dh), jnp.float32),
        ],
    )
    def k(dst2_hbm, u_hbm, z_hbm, agg_hbm, idx2, rows, sems, acc):
        cid = lax.axis_index("c")
        sid = lax.axis_index("s")
        col0 = cid * dh

        def fire_rows(i, b):
            pltpu.async_copy(
                u_hbm.at[pl.ds(sid * et + i * ch, ch), pl.ds(col0, dh)],
                rows[b], sems[b])

        def wait_rows(i, b):
            pltpu.make_async_copy(
                u_hbm.at[pl.ds(sid * et + i * ch, ch), pl.ds(col0, dh)],
                rows[b], sems[b]).wait()

        pltpu.sync_copy(dst2_hbm.at[sid], idx2)

        for r in range(nround):
            j = r * ns + sid

            @pl.when(j < nrc)
            def _():
                pltpu.sync_copy(z_hbm.at[pl.ds(j * rc, rc)],
                                acc.at[pl.ds(j * rc, rc)])

        fire_rows(0, 0)
        plsc.subcore_barrier()

        @pl.loop(0, nit // 2)
        def _(g):
            for b in (0, 1):
                i = g * 2 + b

                @pl.when(i + 1 < nit)
                def _():
                    fire_rows(i + 1, 1 - b)

                wait_rows(i, b)
                pltpu.sync_copy(rows[b], acc.at[idx2.at[i]], add=True)

        if nit % 2:
            i = nit - 1
            wait_rows(i, i % 2)
            pltpu.sync_copy(rows[i % 2], acc.at[idx2.at[i]], add=True)

        plsc.subcore_barrier()

        for r in range(nround):
            j = r * ns + sid

            @pl.when(j < nrc)
            def _():
                pltpu.sync_copy(acc.at[pl.ds(j * rc, rc)],
                                agg_hbm.at[pl.ds(j * rc, rc), pl.ds(col0, dh)])

    return k(dst3, u, zeros_half)


def _node_mlp(x, aggs, w_n1, b_n1, w_n2, b_n2, ln_s, ln_b):
    n, d = x.shape
    bn = 1000
    assert n % bn == 0

    def body(x_ref, *refs):
        a_refs, (w1_ref, b1_ref, w2_ref, b2_ref, s_ref, b_ref, o_ref) = (
            refs[:-7], refs[-7:])
        xb = x_ref[...]
        agg = a_refs[0][...]
        for a in a_refs[1:]:
            agg = agg + a[...]
        w1 = w1_ref[...].astype(jnp.bfloat16)
        z = (jnp.dot(xb.astype(jnp.bfloat16), w1[:d],
                     preferred_element_type=jnp.float32)
             + jnp.dot(agg.astype(jnp.bfloat16), w1[d:],
                       preferred_element_type=jnp.float32)
             + b1_ref[...])
        t = jnp.maximum(z, 0.0)
        u = jnp.dot(t.astype(jnp.bfloat16), w2_ref[...].astype(jnp.bfloat16),
                    preferred_element_type=jnp.float32) + b2_ref[...]
        o_ref[...] = _ln(u + xb, s_ref[...], b_ref[...])

    vec = lambda: pl.BlockSpec((1, d), lambda i: (0, 0))
    blk = lambda: pl.BlockSpec((bn, d), lambda i: (i, 0))
    return pl.pallas_call(
        body,
        grid=(n // bn,),
        in_specs=[blk()] + [blk() for _ in aggs]
                 + [pl.BlockSpec((2 * d, d), lambda i: (0, 0)), vec(),
                    pl.BlockSpec((d, d), lambda i: (0, 0)), vec(), vec(),
                    vec()],
        out_specs=blk(),
        out_shape=jax.ShapeDtypeStruct((n, d), jnp.float32),
    )(x, *aggs, w_n1, b_n1.reshape(1, d), w_n2, b_n2.reshape(1, d),
      ln_s.reshape(1, d), ln_b.reshape(1, d))


def kernel(x, edge_attr, W_e1, b_e1, W_e2, b_e2, W_n1, b_n1, W_n2, b_n2,
           ln_n_scale, ln_n_bias, ln_e_scale, ln_e_bias, edge_index):
    n, d = x.shape
    src = edge_index[0]
    dst = edge_index[1]

    e = src.shape[0]
    w32 = e // 5
    windows = [(0, w32), (w32, 2 * w32), (3 * w32, 2 * w32)]

    sch = 80
    zeros_half = jnp.zeros((n, d // 2), jnp.float32)
    we, be1 = W_e1, b_e1

    ps, pd = _precompute_tables(x, W_e1)
    gs = [_sc_gather_pair(src, dst, ps, pd, w) for w in windows[:2]]
    eo = None
    aggs = []
    for k, w in enumerate(windows):
        u, eo = _edge_mlp(gs[k][0], gs[k][1], edge_attr, we, be1, W_e2, b_e2,
                          ln_e_scale, ln_e_bias, w, eo_seed=eo)
        if len(gs) < len(windows):
            gs.append(_sc_gather_pair(src, dst, ps, pd, windows[len(gs)]))
        dst3 = dst[w[0] : w[0] + w[1]].reshape(16, -1, sch)
        aggs.append(_sc_segment_sum(dst3, u, zeros_half))
    x_out = _node_mlp(x, aggs, W_n1, b_n1, W_n2, b_n2, ln_n_scale, ln_n_bias)
    return x_out, eo
